# Initial kernel scaffold; baseline (speedup 1.0000x reference)
import jax
import jax.numpy as jnp
from jax.experimental import pallas as pl


def kernel(node_embeddings, rel_binary, rel_unary, Wb1, bb1, Wb2, bb2, Wu1, bu1, Wu2, bu2, Wup1, bup1, Wup2, bup2):
    raise NotImplementedError("write your pallas kernel here")


# speedup vs baseline: 2.8365x; 2.8365x over previous
import functools

import jax
import jax.numpy as jnp
from jax import lax
from jax.experimental import pallas as pl
from jax.experimental.pallas import tpu as pltpu
from jax.experimental.pallas import tpu_sc as plsc

EMB = 128
N = 10000
NB = 512000
NU = 100000
NE = NB // 2

CHUNK = 128
WORKERS = 32
ALIGN = CHUNK * WORKERS

NE_P = 262144
NU_P = 102400
TOT_P = 2 * NE_P + NU_P
NPAD = 10240

_vector_mesh = plsc.VectorSubcoreMesh(
    core_axis_name="core", subcore_axis_name="subcore")


def _sc_gather(table, idx2d):
    m = idx2d.shape[1]

    @functools.partial(
        pl.kernel,
        out_type=jax.ShapeDtypeStruct((m, EMB), table.dtype),
        mesh=_vector_mesh,
    )
    def k(x_hbm, i_hbm, o_hbm):
        def body(i_vmem, o_vmem):
            pltpu.sync_copy(x_hbm.at[i_vmem.at[0]], o_vmem)

        pltpu.emit_pipeline(
            body,
            grid=(m // CHUNK,),
            in_specs=[pl.BlockSpec((1, CHUNK), index_map=lambda i: (0, i))],
            out_specs=[pl.BlockSpec((CHUNK, EMB), index_map=lambda i: (i, 0))],
            core_axis_name=("core", "subcore"),
            dimension_semantics=(pltpu.PARALLEL,),
        )(i_hbm, o_hbm)

    return k(table, idx2d)


def _sc_scatter_add(msgs_and_idx, zeros_init):
    @functools.partial(
        pl.kernel,
        out_type=jax.ShapeDtypeStruct((2, NPAD, EMB), jnp.float32),
        mesh=_vector_mesh,
        scratch_types=[pltpu.VMEM_SHARED((NPAD, EMB), jnp.float32)],
    )
    def k(me, mo, mu, ie, io, iu, z_hbm, out_hbm, acc):
        cid = lax.axis_index("core")
        sid = lax.axis_index("subcore")

        @pl.when(sid == 0)
        def _():
            pltpu.sync_copy(z_hbm, acc)

        plsc.subcore_barrier()

        def body(m_vmem, i_vmem):
            pltpu.sync_copy(m_vmem, acc.at[i_vmem.at[0]], add=True)

        for m_hbm, i_hbm in ((me, ie), (mo, io), (mu, iu)):
            g = m_hbm.shape[0] // CHUNK
            pltpu.emit_pipeline(
                body,
                grid=(g,),
                in_specs=[
                    pl.BlockSpec((CHUNK, EMB), index_map=lambda i: (i, 0)),
                    pl.BlockSpec((1, CHUNK), index_map=lambda i: (0, i)),
                ],
                out_specs=[],
                core_axis_name=("core", "subcore"),
                dimension_semantics=(pltpu.PARALLEL,),
            )(m_hbm, i_hbm)

        plsc.subcore_barrier()

        @pl.when(sid == 0)
        def _():
            pltpu.sync_copy(acc, out_hbm.at[cid])

    me, ie = msgs_and_idx[0]
    mo, io = msgs_and_idx[1]
    mu, iu = msgs_and_idx[2]
    return k(me, mo, mu, ie, io, iu, zeros_init)


def _tc_binary_msgs(g_rows, w1a, w1b, b1, w2, b2, blk=512):
    nblk = NE_P // blk

    def body(ge_ref, go_ref, w1a_ref, w1b_ref, b1_ref, w2_ref, b2_ref,
             me_ref, mo_ref):
        ge = ge_ref[...]
        go = go_ref[...]
        h = jnp.dot(ge, w1a_ref[...], preferred_element_type=jnp.float32)
        h += jnp.dot(go, w1b_ref[...], preferred_element_type=jnp.float32)
        h = jnp.maximum(h + b1_ref[...], 0.0)
        y = jnp.dot(h, w2_ref[...], preferred_element_type=jnp.float32)
        y += b2_ref[...]
        me_ref[...] = ge + y[:, :EMB]
        mo_ref[...] = go + y[:, EMB:]

    out_shape = [jax.ShapeDtypeStruct((NE_P, EMB), jnp.float32)] * 2
    return pl.pallas_call(
        body,
        grid=(nblk,),
        in_specs=[
            pl.BlockSpec((blk, EMB), lambda i: (i, 0)),
            pl.BlockSpec((blk, EMB), lambda i, nb=nblk: (i + nb, 0)),
            pl.BlockSpec((EMB, 2 * EMB), lambda i: (0, 0)),
            pl.BlockSpec((EMB, 2 * EMB), lambda i: (0, 0)),
            pl.BlockSpec((1, 2 * EMB), lambda i: (0, 0)),
            pl.BlockSpec((2 * EMB, 2 * EMB), lambda i: (0, 0)),
            pl.BlockSpec((1, 2 * EMB), lambda i: (0, 0)),
        ],
        out_specs=[
            pl.BlockSpec((blk, EMB), lambda i: (i, 0)),
            pl.BlockSpec((blk, EMB), lambda i: (i, 0)),
        ],
        out_shape=out_shape,
    )(g_rows, g_rows, w1a, w1b, b1, w2, b2)


def _tc_unary_msgs(g_rows, w1, b1, w2, b2, blk=512):
    nblk = NU_P // blk
    base = (2 * NE_P) // blk

    def body(gu_ref, w1_ref, b1_ref, w2_ref, b2_ref, mu_ref):
        gu = gu_ref[...]
        h = jnp.dot(gu, w1_ref[...], preferred_element_type=jnp.float32)
        h = jnp.maximum(h + b1_ref[...], 0.0)
        y = jnp.dot(h, w2_ref[...], preferred_element_type=jnp.float32)
        mu_ref[...] = gu + y + b2_ref[...]

    return pl.pallas_call(
        body,
        grid=(nblk,),
        in_specs=[
            pl.BlockSpec((blk, EMB), lambda i, b=base: (i + b, 0)),
            pl.BlockSpec((EMB, EMB), lambda i: (0, 0)),
            pl.BlockSpec((1, EMB), lambda i: (0, 0)),
            pl.BlockSpec((EMB, EMB), lambda i: (0, 0)),
            pl.BlockSpec((1, EMB), lambda i: (0, 0)),
        ],
        out_specs=pl.BlockSpec((blk, EMB), lambda i: (i, 0)),
        out_shape=jax.ShapeDtypeStruct((NU_P, EMB), jnp.float32),
    )(g_rows, w1, b1, w2, b2)


def _tc_update(partials, node_emb, w1, b1, w2, b2, blk=1000):
    nblk = N // blk

    def body(a0_ref, a1_ref, emb_ref, w1_ref, b1_ref, w2_ref, b2_ref, o_ref):
        s = a0_ref[0] + a1_ref[0]
        x = jnp.concatenate([s, emb_ref[...]], axis=1)
        h = jnp.dot(x, w1_ref[...], preferred_element_type=jnp.float32)
        h = jnp.maximum(h + b1_ref[...], 0.0)
        y = jnp.dot(h, w2_ref[...], preferred_element_type=jnp.float32)
        o_ref[...] = y + b2_ref[...]

    return pl.pallas_call(
        body,
        grid=(nblk,),
        in_specs=[
            pl.BlockSpec((1, blk, EMB), lambda i: (0, i, 0)),
            pl.BlockSpec((1, blk, EMB), lambda i: (1, i, 0)),
            pl.BlockSpec((blk, EMB), lambda i: (i, 0)),
            pl.BlockSpec((2 * EMB, 2 * EMB), lambda i: (0, 0)),
            pl.BlockSpec((1, 2 * EMB), lambda i: (0, 0)),
            pl.BlockSpec((2 * EMB, EMB), lambda i: (0, 0)),
            pl.BlockSpec((1, EMB), lambda i: (0, 0)),
        ],
        out_specs=pl.BlockSpec((blk, EMB), lambda i: (i, 0)),
        out_shape=jax.ShapeDtypeStruct((N, EMB), jnp.float32),
    )(partials, partials, node_emb, w1, b1, w2, b2)


def kernel(node_embeddings, rel_binary, rel_unary,
           Wb1, bb1, Wb2, bb2,
           Wu1, bu1, Wu2, bu2,
           Wup1, bup1, Wup2, bup2):
    idx_be = rel_binary[0::2]
    idx_bo = rel_binary[1::2]

    zero_i = jnp.zeros((NE_P - NE,), jnp.int32)
    zero_u = jnp.zeros((NU_P - NU,), jnp.int32)
    gather_idx = jnp.concatenate(
        [idx_be, zero_i, idx_bo, zero_i, rel_unary, zero_u]).reshape(1, TOT_P)

    dummy_i = jnp.full((NE_P - NE,), N, jnp.int32)
    dummy_u = jnp.full((NU_P - NU,), N, jnp.int32)
    ie = jnp.concatenate([idx_be, dummy_i]).reshape(1, NE_P)
    io = jnp.concatenate([idx_bo, dummy_i]).reshape(1, NE_P)
    iu = jnp.concatenate([rel_unary, dummy_u]).reshape(1, NU_P)

    g_rows = _sc_gather(node_embeddings, gather_idx)

    me, mo = _tc_binary_msgs(
        g_rows, Wb1[:EMB], Wb1[EMB:], bb1.reshape(1, -1), Wb2,
        bb2.reshape(1, -1))
    mu = _tc_unary_msgs(
        g_rows, Wu1, bu1.reshape(1, -1), Wu2, bu2.reshape(1, -1))

    zeros_init = jnp.zeros((NPAD, EMB), jnp.float32)
    partials = _sc_scatter_add(
        ((me, ie), (mo, io), (mu, iu)), zeros_init)

    return _tc_update(
        partials, node_embeddings, Wup1, bup1.reshape(1, -1), Wup2,
        bup2.reshape(1, -1))


# speedup vs baseline: 2.9746x; 1.0487x over previous
import functools

import jax
import jax.numpy as jnp
from jax import lax
from jax.experimental import pallas as pl
from jax.experimental.pallas import tpu as pltpu
from jax.experimental.pallas import tpu_sc as plsc

EMB = 128
HALF = EMB // 2
N = 10000
NB = 512000
NU = 100000
NE = NB // 2

CHUNK = 128
GB = 384
WORKERS = 32

NE_P = 262144
NU_P = 102400
TOT_P = 2 * NE_P + NU_P
TOT_G = TOT_P
NPAD = 10240

_vector_mesh = plsc.VectorSubcoreMesh(
    core_axis_name="core", subcore_axis_name="subcore")


def _sc_gather(table, idx2d):
    m = idx2d.shape[1]

    @functools.partial(
        pl.kernel,
        out_type=jax.ShapeDtypeStruct((m, EMB), jnp.float32),
        mesh=_vector_mesh,
        scratch_types=[pltpu.SemaphoreType.DMA],
    )
    def k(x_hbm, i_hbm, o_hbm, sem):
        def body(i_vmem, o_vmem):
            copies = [
                pltpu.async_copy(
                    x_hbm.at[i_vmem.at[0, pl.ds(c * CHUNK, CHUNK)]],
                    o_vmem.at[pl.ds(c * CHUNK, CHUNK)], sem)
                for c in range(GB // CHUNK)
            ]
            for c in copies:
                c.wait()

        pltpu.emit_pipeline(
            body,
            grid=(m // GB,),
            in_specs=[pl.BlockSpec((1, GB), index_map=lambda i: (0, i))],
            out_specs=[pl.BlockSpec((GB, EMB), index_map=lambda i: (i, 0))],
            core_axis_name=("core", "subcore"),
            dimension_semantics=(pltpu.PARALLEL,),
        )(i_hbm, o_hbm)

    return k(table, idx2d)


def _sc_scatter_add(msgs_and_idx, zeros_init):
    @functools.partial(
        pl.kernel,
        out_type=jax.ShapeDtypeStruct((2, NPAD, EMB), jnp.float32),
        mesh=_vector_mesh,
        scratch_types=[pltpu.VMEM_SHARED((NPAD, EMB), jnp.float32)],
    )
    def k(me, mo, mu, ie, io, iu, z_hbm, out_hbm, acc):
        cid = lax.axis_index("core")
        sid = lax.axis_index("subcore")

        @pl.when(sid == 0)
        def _():
            pltpu.sync_copy(z_hbm, acc)

        plsc.subcore_barrier()

        def body(m_vmem, i_vmem):
            pltpu.sync_copy(m_vmem, acc.at[i_vmem.at[0]], add=True)

        for m_hbm, i_hbm in ((me, ie), (mo, io), (mu, iu)):
            g = m_hbm.shape[0] // CHUNK
            pltpu.emit_pipeline(
                body,
                grid=(g,),
                in_specs=[
                    pl.BlockSpec((CHUNK, EMB), index_map=lambda i: (i, 0)),
                    pl.BlockSpec((1, CHUNK), index_map=lambda i: (0, i)),
                ],
                out_specs=[],
                core_axis_name=("core", "subcore"),
                dimension_semantics=(pltpu.PARALLEL,),
            )(m_hbm, i_hbm)

        plsc.subcore_barrier()

        @pl.when(sid == 0)
        def _():
            pltpu.sync_copy(acc, out_hbm.at[cid])

    me, ie = msgs_and_idx[0]
    mo, io = msgs_and_idx[1]
    mu, iu = msgs_and_idx[2]
    return k(me, mo, mu, ie, io, iu, zeros_init)


def _tc_binary_msgs(g_rows, w1a, w1b, b1, w2, b2, blk=512):
    nblk = NE_P // blk

    def body(ge_ref, go_ref, w1a_ref, w1b_ref, b1_ref, w2_ref, b2_ref,
             me_ref, mo_ref):
        ge = ge_ref[...]
        go = go_ref[...]
        h = jnp.dot(ge.astype(jnp.bfloat16), w1a_ref[...],
                    preferred_element_type=jnp.float32)
        h += jnp.dot(go.astype(jnp.bfloat16), w1b_ref[...],
                     preferred_element_type=jnp.float32)
        h = jnp.maximum(h + b1_ref[...], 0.0).astype(jnp.bfloat16)
        y = jnp.dot(h, w2_ref[...], preferred_element_type=jnp.float32)
        y += b2_ref[...]
        me_ref[...] = ge + y[:, :EMB]
        mo_ref[...] = go + y[:, EMB:]

    out_shape = [jax.ShapeDtypeStruct((NE_P, EMB), jnp.float32)] * 2
    return pl.pallas_call(
        body,
        grid=(nblk,),
        in_specs=[
            pl.BlockSpec((blk, EMB), lambda i: (i, 0)),
            pl.BlockSpec((blk, EMB), lambda i, nb=nblk: (i + nb, 0)),
            pl.BlockSpec((EMB, 2 * EMB), lambda i: (0, 0)),
            pl.BlockSpec((EMB, 2 * EMB), lambda i: (0, 0)),
            pl.BlockSpec((1, 2 * EMB), lambda i: (0, 0)),
            pl.BlockSpec((2 * EMB, 2 * EMB), lambda i: (0, 0)),
            pl.BlockSpec((1, 2 * EMB), lambda i: (0, 0)),
        ],
        out_specs=[
            pl.BlockSpec((blk, EMB), lambda i: (i, 0)),
            pl.BlockSpec((blk, EMB), lambda i: (i, 0)),
        ],
        out_shape=out_shape,
    )(g_rows, g_rows, w1a, w1b, b1, w2, b2)


def _tc_unary_msgs(g_rows, w1, b1, w2, b2, blk=512):
    nblk = NU_P // blk
    base = (2 * NE_P) // blk

    def body(gu_ref, w1_ref, b1_ref, w2_ref, b2_ref, mu_ref):
        gu = gu_ref[...]
        h = jnp.dot(gu.astype(jnp.bfloat16), w1_ref[...],
                    preferred_element_type=jnp.float32)
        h = jnp.maximum(h + b1_ref[...], 0.0).astype(jnp.bfloat16)
        y = jnp.dot(h, w2_ref[...], preferred_element_type=jnp.float32)
        mu_ref[...] = gu + y + b2_ref[...]

    return pl.pallas_call(
        body,
        grid=(nblk,),
        in_specs=[
            pl.BlockSpec((blk, EMB), lambda i, b=base: (i + b, 0)),
            pl.BlockSpec((EMB, EMB), lambda i: (0, 0)),
            pl.BlockSpec((1, EMB), lambda i: (0, 0)),
            pl.BlockSpec((EMB, EMB), lambda i: (0, 0)),
            pl.BlockSpec((1, EMB), lambda i: (0, 0)),
        ],
        out_specs=pl.BlockSpec((blk, EMB), lambda i: (i, 0)),
        out_shape=jax.ShapeDtypeStruct((NU_P, EMB), jnp.float32),
    )(g_rows, w1, b1, w2, b2)


def _tc_update(partials, node_emb, w1, b1, w2, b2, blk=1000):
    nblk = N // blk

    def body(a0_ref, a1_ref, emb_ref, w1_ref, b1_ref, w2_ref, b2_ref, o_ref):
        s = a0_ref[0] + a1_ref[0]
        x = jnp.concatenate([s, emb_ref[...]], axis=1).astype(jnp.bfloat16)
        h = jnp.dot(x, w1_ref[...], preferred_element_type=jnp.float32)
        h = jnp.maximum(h + b1_ref[...], 0.0).astype(jnp.bfloat16)
        y = jnp.dot(h, w2_ref[...], preferred_element_type=jnp.float32)
        o_ref[...] = y + b2_ref[...]

    return pl.pallas_call(
        body,
        grid=(nblk,),
        in_specs=[
            pl.BlockSpec((1, blk, EMB), lambda i: (0, i, 0)),
            pl.BlockSpec((1, blk, EMB), lambda i: (1, i, 0)),
            pl.BlockSpec((blk, EMB), lambda i: (i, 0)),
            pl.BlockSpec((2 * EMB, 2 * EMB), lambda i: (0, 0)),
            pl.BlockSpec((1, 2 * EMB), lambda i: (0, 0)),
            pl.BlockSpec((2 * EMB, EMB), lambda i: (0, 0)),
            pl.BlockSpec((1, EMB), lambda i: (0, 0)),
        ],
        out_specs=pl.BlockSpec((blk, EMB), lambda i: (i, 0)),
        out_shape=jax.ShapeDtypeStruct((N, EMB), jnp.float32),
    )(partials, partials, node_emb, w1, b1, w2, b2)


def kernel(node_embeddings, rel_binary, rel_unary,
           Wb1, bb1, Wb2, bb2,
           Wu1, bu1, Wu2, bu2,
           Wup1, bup1, Wup2, bup2):
    bf = jnp.bfloat16
    idx_be = rel_binary[0::2]
    idx_bo = rel_binary[1::2]

    zero_i = jnp.zeros((NE_P - NE,), jnp.int32)
    zero_u = jnp.zeros((NU_P - NU,), jnp.int32)
    gather_idx = jnp.concatenate(
        [idx_be, zero_i, idx_bo, zero_i, rel_unary, zero_u]).reshape(1, TOT_G)

    dummy_i = jnp.full((NE_P - NE,), N, jnp.int32)
    dummy_u = jnp.full((NU_P - NU,), N, jnp.int32)
    ie = jnp.concatenate([idx_be, dummy_i]).reshape(1, NE_P)
    io = jnp.concatenate([idx_bo, dummy_i]).reshape(1, NE_P)
    iu = jnp.concatenate([rel_unary, dummy_u]).reshape(1, NU_P)

    g_rows = _sc_gather(node_embeddings, gather_idx)

    me, mo = _tc_binary_msgs(
        g_rows, Wb1[:EMB].astype(bf), Wb1[EMB:].astype(bf),
        bb1.reshape(1, -1), Wb2.astype(bf), bb2.reshape(1, -1))
    mu = _tc_unary_msgs(
        g_rows, Wu1.astype(bf), bu1.reshape(1, -1), Wu2.astype(bf),
        bu2.reshape(1, -1))

    zeros_init = jnp.zeros((NPAD, EMB), jnp.float32)
    partials = _sc_scatter_add(
        ((me, ie), (mo, io), (mu, iu)), zeros_init)

    return _tc_update(
        partials, node_embeddings, Wup1.astype(bf), bup1.reshape(1, -1),
        Wup2.astype(bf), bup2.reshape(1, -1))


# speedup vs baseline: 4.9953x; 1.6793x over previous
import functools

import jax
import jax.numpy as jnp
from jax import lax
from jax.experimental import pallas as pl
from jax.experimental.pallas import tpu as pltpu
from jax.experimental.pallas import tpu_sc as plsc

EMB = 128
HALF = EMB // 2
N = 10000
NB = 512000
NU = 100000
NE = NB // 2

CHUNK = 128
GB = 128
WORKERS = 32

NE_P = 262144
NU_P = 102400
TOT_P = 2 * NE_P + NU_P
TOT_G = TOT_P
NPAD = 10240

_vector_mesh = plsc.VectorSubcoreMesh(
    core_axis_name="core", subcore_axis_name="subcore")


def _sc_gather(table, idx2d):
    m = idx2d.shape[1]

    @functools.partial(
        pl.kernel,
        out_type=jax.ShapeDtypeStruct((m, EMB), jnp.float32),
        mesh=_vector_mesh,
        scratch_types=[pltpu.SemaphoreType.DMA,
                       pltpu.VMEM_SHARED((N, EMB), jnp.float32),
                       pltpu.VMEM((64, EMB), jnp.float32)],
    )
    def k(x_hbm, i_hbm, o_hbm, sem, table_sp, stage):
        sid = lax.axis_index("subcore")
        nfull = N // 64

        @pl.loop(0, 10)
        def _(j):
            c = sid + j * 16

            @pl.when(c < nfull)
            def _():
                off = pl.multiple_of(c * 64, 64)
                pltpu.sync_copy(x_hbm.at[pl.ds(off, 64)], stage)
                pltpu.sync_copy(stage, table_sp.at[pl.ds(off, 64)])

        @pl.when(sid == 15)
        def _():
            tail = nfull * 64
            pltpu.sync_copy(x_hbm.at[pl.ds(tail, N - tail)],
                            stage.at[pl.ds(0, N - tail)])
            pltpu.sync_copy(stage.at[pl.ds(0, N - tail)],
                            table_sp.at[pl.ds(tail, N - tail)])

        plsc.subcore_barrier()

        def body(i_vmem, o_vmem):
            copies = [
                pltpu.async_copy(
                    table_sp.at[i_vmem.at[0, pl.ds(c * CHUNK, CHUNK)]],
                    o_vmem.at[pl.ds(c * CHUNK, CHUNK)], sem)
                for c in range(GB // CHUNK)
            ]
            for c in copies:
                c.wait()

        pltpu.emit_pipeline(
            body,
            grid=(m // GB,),
            in_specs=[pl.BlockSpec((1, GB), index_map=lambda i: (0, i))],
            out_specs=[pl.BlockSpec((GB, EMB), index_map=lambda i: (i, 0))],
            core_axis_name=("core", "subcore"),
            dimension_semantics=(pltpu.PARALLEL,),
        )(i_hbm, o_hbm)

    return k(table, idx2d)


def _sc_scatter_add(msgs_and_idx, zeros_init):
    @functools.partial(
        pl.kernel,
        out_type=jax.ShapeDtypeStruct((2, NPAD, EMB), jnp.float32),
        mesh=_vector_mesh,
        scratch_types=[pltpu.VMEM_SHARED((NPAD, EMB), jnp.float32)],
    )
    def k(me, mo, mu, ie, io, iu, z_hbm, out_hbm, acc):
        cid = lax.axis_index("core")
        sid = lax.axis_index("subcore")

        @pl.when(sid == 0)
        def _():
            pltpu.sync_copy(z_hbm, acc)

        plsc.subcore_barrier()

        def body(m_vmem, i_vmem):
            pltpu.sync_copy(m_vmem, acc.at[i_vmem.at[0]], add=True)

        for m_hbm, i_hbm in ((me, ie), (mo, io), (mu, iu)):
            g = m_hbm.shape[0] // CHUNK
            pltpu.emit_pipeline(
                body,
                grid=(g,),
                in_specs=[
                    pl.BlockSpec((CHUNK, EMB), index_map=lambda i: (i, 0)),
                    pl.BlockSpec((1, CHUNK), index_map=lambda i: (0, i)),
                ],
                out_specs=[],
                core_axis_name=("core", "subcore"),
                dimension_semantics=(pltpu.PARALLEL,),
            )(m_hbm, i_hbm)

        plsc.subcore_barrier()

        @pl.when(sid == 0)
        def _():
            pltpu.sync_copy(acc, out_hbm.at[cid])

    me, ie = msgs_and_idx[0]
    mo, io = msgs_and_idx[1]
    mu, iu = msgs_and_idx[2]
    return k(me, mo, mu, ie, io, iu, zeros_init)


def _tc_binary_msgs(g_rows, w1a, w1b, b1, w2, b2, blk=512):
    nblk = NE_P // blk

    def body(ge_ref, go_ref, w1a_ref, w1b_ref, b1_ref, w2_ref, b2_ref,
             me_ref, mo_ref):
        ge = ge_ref[...]
        go = go_ref[...]
        h = jnp.dot(ge.astype(jnp.bfloat16), w1a_ref[...],
                    preferred_element_type=jnp.float32)
        h += jnp.dot(go.astype(jnp.bfloat16), w1b_ref[...],
                     preferred_element_type=jnp.float32)
        h = jnp.maximum(h + b1_ref[...], 0.0).astype(jnp.bfloat16)
        y = jnp.dot(h, w2_ref[...], preferred_element_type=jnp.float32)
        y += b2_ref[...]
        me_ref[...] = ge + y[:, :EMB]
        mo_ref[...] = go + y[:, EMB:]

    out_shape = [jax.ShapeDtypeStruct((NE_P, EMB), jnp.float32)] * 2
    return pl.pallas_call(
        body,
        grid=(nblk,),
        in_specs=[
            pl.BlockSpec((blk, EMB), lambda i: (i, 0)),
            pl.BlockSpec((blk, EMB), lambda i, nb=nblk: (i + nb, 0)),
            pl.BlockSpec((EMB, 2 * EMB), lambda i: (0, 0)),
            pl.BlockSpec((EMB, 2 * EMB), lambda i: (0, 0)),
            pl.BlockSpec((1, 2 * EMB), lambda i: (0, 0)),
            pl.BlockSpec((2 * EMB, 2 * EMB), lambda i: (0, 0)),
            pl.BlockSpec((1, 2 * EMB), lambda i: (0, 0)),
        ],
        out_specs=[
            pl.BlockSpec((blk, EMB), lambda i: (i, 0)),
            pl.BlockSpec((blk, EMB), lambda i: (i, 0)),
        ],
        out_shape=out_shape,
    )(g_rows, g_rows, w1a, w1b, b1, w2, b2)


def _tc_unary_msgs(g_rows, w1, b1, w2, b2, blk=512):
    nblk = NU_P // blk
    base = (2 * NE_P) // blk

    def body(gu_ref, w1_ref, b1_ref, w2_ref, b2_ref, mu_ref):
        gu = gu_ref[...]
        h = jnp.dot(gu.astype(jnp.bfloat16), w1_ref[...],
                    preferred_element_type=jnp.float32)
        h = jnp.maximum(h + b1_ref[...], 0.0).astype(jnp.bfloat16)
        y = jnp.dot(h, w2_ref[...], preferred_element_type=jnp.float32)
        mu_ref[...] = gu + y + b2_ref[...]

    return pl.pallas_call(
        body,
        grid=(nblk,),
        in_specs=[
            pl.BlockSpec((blk, EMB), lambda i, b=base: (i + b, 0)),
            pl.BlockSpec((EMB, EMB), lambda i: (0, 0)),
            pl.BlockSpec((1, EMB), lambda i: (0, 0)),
            pl.BlockSpec((EMB, EMB), lambda i: (0, 0)),
            pl.BlockSpec((1, EMB), lambda i: (0, 0)),
        ],
        out_specs=pl.BlockSpec((blk, EMB), lambda i: (i, 0)),
        out_shape=jax.ShapeDtypeStruct((NU_P, EMB), jnp.float32),
    )(g_rows, w1, b1, w2, b2)


def _tc_update(partials, node_emb, w1, b1, w2, b2, blk=1000):
    nblk = N // blk

    def body(a0_ref, a1_ref, emb_ref, w1_ref, b1_ref, w2_ref, b2_ref, o_ref):
        s = a0_ref[0] + a1_ref[0]
        x = jnp.concatenate([s, emb_ref[...]], axis=1).astype(jnp.bfloat16)
        h = jnp.dot(x, w1_ref[...], preferred_element_type=jnp.float32)
        h = jnp.maximum(h + b1_ref[...], 0.0).astype(jnp.bfloat16)
        y = jnp.dot(h, w2_ref[...], preferred_element_type=jnp.float32)
        o_ref[...] = y + b2_ref[...]

    return pl.pallas_call(
        body,
        grid=(nblk,),
        in_specs=[
            pl.BlockSpec((1, blk, EMB), lambda i: (0, i, 0)),
            pl.BlockSpec((1, blk, EMB), lambda i: (1, i, 0)),
            pl.BlockSpec((blk, EMB), lambda i: (i, 0)),
            pl.BlockSpec((2 * EMB, 2 * EMB), lambda i: (0, 0)),
            pl.BlockSpec((1, 2 * EMB), lambda i: (0, 0)),
            pl.BlockSpec((2 * EMB, EMB), lambda i: (0, 0)),
            pl.BlockSpec((1, EMB), lambda i: (0, 0)),
        ],
        out_specs=pl.BlockSpec((blk, EMB), lambda i: (i, 0)),
        out_shape=jax.ShapeDtypeStruct((N, EMB), jnp.float32),
    )(partials, partials, node_emb, w1, b1, w2, b2)


def kernel(node_embeddings, rel_binary, rel_unary,
           Wb1, bb1, Wb2, bb2,
           Wu1, bu1, Wu2, bu2,
           Wup1, bup1, Wup2, bup2):
    bf = jnp.bfloat16
    idx_be = rel_binary[0::2]
    idx_bo = rel_binary[1::2]

    zero_i = jnp.zeros((NE_P - NE,), jnp.int32)
    zero_u = jnp.zeros((NU_P - NU,), jnp.int32)
    gather_idx = jnp.concatenate(
        [idx_be, zero_i, idx_bo, zero_i, rel_unary, zero_u]).reshape(1, TOT_G)

    dummy_i = jnp.full((NE_P - NE,), N, jnp.int32)
    dummy_u = jnp.full((NU_P - NU,), N, jnp.int32)
    ie = jnp.concatenate([idx_be, dummy_i]).reshape(1, NE_P)
    io = jnp.concatenate([idx_bo, dummy_i]).reshape(1, NE_P)
    iu = jnp.concatenate([rel_unary, dummy_u]).reshape(1, NU_P)

    g_rows = _sc_gather(node_embeddings, gather_idx)

    me, mo = _tc_binary_msgs(
        g_rows, Wb1[:EMB].astype(bf), Wb1[EMB:].astype(bf),
        bb1.reshape(1, -1), Wb2.astype(bf), bb2.reshape(1, -1))
    mu = _tc_unary_msgs(
        g_rows, Wu1.astype(bf), bu1.reshape(1, -1), Wu2.astype(bf),
        bu2.reshape(1, -1))

    zeros_init = jnp.zeros((NPAD, EMB), jnp.float32)
    partials = _sc_scatter_add(
        ((me, ie), (mo, io), (mu, iu)), zeros_init)

    return _tc_update(
        partials, node_embeddings, Wup1.astype(bf), bup1.reshape(1, -1),
        Wup2.astype(bf), bup2.reshape(1, -1))


# speedup vs baseline: 7.6718x; 1.5358x over previous
import functools

import jax
import jax.numpy as jnp
from jax import lax
from jax.experimental import pallas as pl
from jax.experimental.pallas import tpu as pltpu
from jax.experimental.pallas import tpu_sc as plsc

EMB = 128
N = 10000
NB = 512000
NU = 100000
NE = NB // 2

CHUNK = 128
H = 131072
NU_P = 102400
NPAD = 10240

_vector_mesh = plsc.VectorSubcoreMesh(
    core_axis_name="core", subcore_axis_name="subcore")


def _sc_gather(table, idx2d):
    m = idx2d.shape[1]

    @functools.partial(
        pl.kernel,
        out_type=jax.ShapeDtypeStruct((m, EMB), jnp.float32),
        mesh=_vector_mesh,
        scratch_types=[pltpu.SemaphoreType.DMA,
                       pltpu.VMEM_SHARED((N, EMB), jnp.float32),
                       pltpu.VMEM((64, EMB), jnp.float32)],
    )
    def k(x_hbm, i_hbm, o_hbm, sem, table_sp, stage):
        sid = lax.axis_index("subcore")
        nfull = N // 64

        @pl.loop(0, 10)
        def _(j):
            c = sid + j * 16

            @pl.when(c < nfull)
            def _():
                off = pl.multiple_of(c * 64, 64)
                pltpu.sync_copy(x_hbm.at[pl.ds(off, 64)], stage)
                pltpu.sync_copy(stage, table_sp.at[pl.ds(off, 64)])

        @pl.when(sid == 15)
        def _():
            tail = nfull * 64
            pltpu.sync_copy(x_hbm.at[pl.ds(tail, N - tail)],
                            stage.at[pl.ds(0, N - tail)])
            pltpu.sync_copy(stage.at[pl.ds(0, N - tail)],
                            table_sp.at[pl.ds(tail, N - tail)])

        plsc.subcore_barrier()

        def body(i_vmem, o_vmem):
            pltpu.async_copy(table_sp.at[i_vmem.at[0]], o_vmem, sem).wait()

        pltpu.emit_pipeline(
            body,
            grid=(m // CHUNK,),
            in_specs=[pl.BlockSpec((1, CHUNK), index_map=lambda i: (0, i))],
            out_specs=[pl.BlockSpec((CHUNK, EMB), index_map=lambda i: (i, 0))],
            core_axis_name=("core", "subcore"),
            dimension_semantics=(pltpu.PARALLEL,),
        )(i_hbm, o_hbm)

    return k(table, idx2d)


def _sc_scatter_add(streams, zeros_init):
    nstream = len(streams)

    @functools.partial(
        pl.kernel,
        out_type=jax.ShapeDtypeStruct((2, NPAD, EMB), jnp.float32),
        mesh=_vector_mesh,
        scratch_types=[pltpu.VMEM_SHARED((NPAD, EMB), jnp.float32)],
    )
    def k(*args):
        refs = args[:2 * nstream]
        z_hbm = args[2 * nstream]
        out_hbm = args[2 * nstream + 1]
        acc = args[2 * nstream + 2]
        cid = lax.axis_index("core")
        sid = lax.axis_index("subcore")

        @pl.when(sid == 0)
        def _():
            pltpu.sync_copy(z_hbm, acc)

        plsc.subcore_barrier()

        def body(m_vmem, i_vmem):
            pltpu.sync_copy(m_vmem, acc.at[i_vmem.at[0]], add=True)

        for s in range(nstream):
            m_hbm, i_hbm = refs[s], refs[nstream + s]
            pltpu.emit_pipeline(
                body,
                grid=(m_hbm.shape[0] // CHUNK,),
                in_specs=[
                    pl.BlockSpec((CHUNK, EMB), index_map=lambda i: (i, 0)),
                    pl.BlockSpec((1, CHUNK), index_map=lambda i: (0, i)),
                ],
                out_specs=[],
                core_axis_name=("core", "subcore"),
                dimension_semantics=(pltpu.PARALLEL,),
            )(m_hbm, i_hbm)

        plsc.subcore_barrier()

        @pl.when(sid == 0)
        def _():
            pltpu.sync_copy(acc, out_hbm.at[cid])

    msgs = [s[0] for s in streams]
    idxs = [s[1] for s in streams]
    return k(*msgs, *idxs, zeros_init)


def _tc_binary_msgs(g, w1a, w1b, b1, w2, b2, blk=1024):
    nblk = H // blk

    def body(ge_ref, go_ref, w1a_ref, w1b_ref, b1_ref, w2_ref, b2_ref,
             me_ref, mo_ref):
        ge = ge_ref[...]
        go = go_ref[...]
        h = jnp.dot(ge.astype(jnp.bfloat16), w1a_ref[...],
                    preferred_element_type=jnp.float32)
        h += jnp.dot(go.astype(jnp.bfloat16), w1b_ref[...],
                     preferred_element_type=jnp.float32)
        h = jnp.maximum(h + b1_ref[...], 0.0).astype(jnp.bfloat16)
        y = jnp.dot(h, w2_ref[...], preferred_element_type=jnp.float32)
        y += b2_ref[...]
        me_ref[...] = ge + y[:, :EMB]
        mo_ref[...] = go + y[:, EMB:]

    out_shape = [jax.ShapeDtypeStruct((H, EMB), jnp.float32)] * 2
    return pl.pallas_call(
        body,
        grid=(nblk,),
        in_specs=[
            pl.BlockSpec((blk, EMB), lambda i: (i, 0)),
            pl.BlockSpec((blk, EMB), lambda i, nb=nblk: (i + nb, 0)),
            pl.BlockSpec((EMB, 2 * EMB), lambda i: (0, 0)),
            pl.BlockSpec((EMB, 2 * EMB), lambda i: (0, 0)),
            pl.BlockSpec((1, 2 * EMB), lambda i: (0, 0)),
            pl.BlockSpec((2 * EMB, 2 * EMB), lambda i: (0, 0)),
            pl.BlockSpec((1, 2 * EMB), lambda i: (0, 0)),
        ],
        out_specs=[
            pl.BlockSpec((blk, EMB), lambda i: (i, 0)),
            pl.BlockSpec((blk, EMB), lambda i: (i, 0)),
        ],
        out_shape=out_shape,
    )(g, g, w1a, w1b, b1, w2, b2)


def _tc_unary_msgs(g, w1, b1, w2, b2, blk=1024):
    nblk = NU_P // blk

    def body(gu_ref, w1_ref, b1_ref, w2_ref, b2_ref, mu_ref):
        gu = gu_ref[...]
        h = jnp.dot(gu.astype(jnp.bfloat16), w1_ref[...],
                    preferred_element_type=jnp.float32)
        h = jnp.maximum(h + b1_ref[...], 0.0).astype(jnp.bfloat16)
        y = jnp.dot(h, w2_ref[...], preferred_element_type=jnp.float32)
        mu_ref[...] = gu + y + b2_ref[...]

    return pl.pallas_call(
        body,
        grid=(nblk,),
        in_specs=[
            pl.BlockSpec((blk, EMB), lambda i: (i, 0)),
            pl.BlockSpec((EMB, EMB), lambda i: (0, 0)),
            pl.BlockSpec((1, EMB), lambda i: (0, 0)),
            pl.BlockSpec((EMB, EMB), lambda i: (0, 0)),
            pl.BlockSpec((1, EMB), lambda i: (0, 0)),
        ],
        out_specs=pl.BlockSpec((blk, EMB), lambda i: (i, 0)),
        out_shape=jax.ShapeDtypeStruct((NU_P, EMB), jnp.float32),
    )(g, w1, b1, w2, b2)


def _tc_update(p1, p2, p3, node_emb, w1, b1, w2, b2, blk=1000):
    nblk = N // blk

    def body(a0, a1, b0, b1r, c0, c1, emb_ref, w1_ref, b1_ref, w2_ref,
             b2_ref, o_ref):
        s = a0[0] + a1[0] + b0[0] + b1r[0] + c0[0] + c1[0]
        x = jnp.concatenate([s, emb_ref[...]], axis=1).astype(jnp.bfloat16)
        h = jnp.dot(x, w1_ref[...], preferred_element_type=jnp.float32)
        h = jnp.maximum(h + b1_ref[...], 0.0).astype(jnp.bfloat16)
        y = jnp.dot(h, w2_ref[...], preferred_element_type=jnp.float32)
        o_ref[...] = y + b2_ref[...]

    part_spec0 = pl.BlockSpec((1, blk, EMB), lambda i: (0, i, 0))
    part_spec1 = pl.BlockSpec((1, blk, EMB), lambda i: (1, i, 0))
    return pl.pallas_call(
        body,
        grid=(nblk,),
        in_specs=[
            part_spec0, part_spec1,
            part_spec0, part_spec1,
            part_spec0, part_spec1,
            pl.BlockSpec((blk, EMB), lambda i: (i, 0)),
            pl.BlockSpec((2 * EMB, 2 * EMB), lambda i: (0, 0)),
            pl.BlockSpec((1, 2 * EMB), lambda i: (0, 0)),
            pl.BlockSpec((2 * EMB, EMB), lambda i: (0, 0)),
            pl.BlockSpec((1, EMB), lambda i: (0, 0)),
        ],
        out_specs=pl.BlockSpec((blk, EMB), lambda i: (i, 0)),
        out_shape=jax.ShapeDtypeStruct((N, EMB), jnp.float32),
    )(p1, p1, p2, p2, p3, p3, node_emb, w1, b1, w2, b2)


def kernel(node_embeddings, rel_binary, rel_unary,
           Wb1, bb1, Wb2, bb2,
           Wu1, bu1, Wu2, bu2,
           Wup1, bup1, Wup2, bup2):
    bf = jnp.bfloat16
    idx_be = rel_binary[0::2]
    idx_bo = rel_binary[1::2]

    pad_b = 2 * H - NE
    pad_u = NU_P - NU
    zero_b = jnp.zeros((pad_b,), jnp.int32)
    zero_u = jnp.zeros((pad_u,), jnp.int32)
    dummy_b = jnp.full((pad_b,), N, jnp.int32)
    dummy_u = jnp.full((pad_u,), N, jnp.int32)

    ig1 = jnp.concatenate([idx_be[:H], idx_bo[:H]]).reshape(1, 2 * H)
    ie1 = idx_be[:H].reshape(1, H)
    io1 = idx_bo[:H].reshape(1, H)
    ig2 = jnp.concatenate(
        [idx_be[H:], zero_b, idx_bo[H:], zero_b]).reshape(1, 2 * H)
    ie2 = jnp.concatenate([idx_be[H:], dummy_b]).reshape(1, H)
    io2 = jnp.concatenate([idx_bo[H:], dummy_b]).reshape(1, H)
    igu = jnp.concatenate([rel_unary, zero_u]).reshape(1, NU_P)
    iu = jnp.concatenate([rel_unary, dummy_u]).reshape(1, NU_P)

    g1 = _sc_gather(node_embeddings, ig1)
    g2 = _sc_gather(node_embeddings, ig2)
    gu = _sc_gather(node_embeddings, igu)

    w1a, w1b = Wb1[:EMB].astype(bf), Wb1[EMB:].astype(bf)
    bb1r, bb2r = bb1.reshape(1, -1), bb2.reshape(1, -1)
    wb2 = Wb2.astype(bf)
    me1, mo1 = _tc_binary_msgs(g1, w1a, w1b, bb1r, wb2, bb2r)
    me2, mo2 = _tc_binary_msgs(g2, w1a, w1b, bb1r, wb2, bb2r)
    mu = _tc_unary_msgs(
        gu, Wu1.astype(bf), bu1.reshape(1, -1), Wu2.astype(bf),
        bu2.reshape(1, -1))

    zeros_init = jnp.zeros((NPAD, EMB), jnp.float32)
    p1 = _sc_scatter_add(((me1, ie1), (mo1, io1)), zeros_init)
    p2 = _sc_scatter_add(((me2, ie2), (mo2, io2)), zeros_init)
    p3 = _sc_scatter_add(((mu, iu),), zeros_init)

    return _tc_update(
        p1, p2, p3, node_embeddings, Wup1.astype(bf), bup1.reshape(1, -1),
        Wup2.astype(bf), bup2.reshape(1, -1))


# speedup vs baseline: 8.3830x; 1.0927x over previous
import functools

import jax
import jax.numpy as jnp
from jax import lax
from jax.experimental import pallas as pl
from jax.experimental.pallas import tpu as pltpu
from jax.experimental.pallas import tpu_sc as plsc

EMB = 128
N = 10000
NB = 512000
NU = 100000
NE = NB // 2

CHUNK = 128
H = 131072
NU_P = 102400
NPAD = 10240

_vector_mesh = plsc.VectorSubcoreMesh(
    core_axis_name="core", subcore_axis_name="subcore")


def _sc_gather(table, idx2d):
    m = idx2d.shape[1]

    @functools.partial(
        pl.kernel,
        out_type=jax.ShapeDtypeStruct((m, EMB), jnp.float32),
        mesh=_vector_mesh,
        scratch_types=[pltpu.SemaphoreType.DMA,
                       pltpu.VMEM_SHARED((N, EMB), jnp.float32),
                       pltpu.VMEM((64, EMB), jnp.float32)],
    )
    def k(x_hbm, i_hbm, o_hbm, sem, table_sp, stage):
        sid = lax.axis_index("subcore")
        nfull = N // 64

        @pl.loop(0, 10)
        def _(j):
            c = sid + j * 16

            @pl.when(c < nfull)
            def _():
                off = pl.multiple_of(c * 64, 64)
                pltpu.sync_copy(x_hbm.at[pl.ds(off, 64)], stage)
                pltpu.sync_copy(stage, table_sp.at[pl.ds(off, 64)])

        @pl.when(sid == 15)
        def _():
            tail = nfull * 64
            pltpu.sync_copy(x_hbm.at[pl.ds(tail, N - tail)],
                            stage.at[pl.ds(0, N - tail)])
            pltpu.sync_copy(stage.at[pl.ds(0, N - tail)],
                            table_sp.at[pl.ds(tail, N - tail)])

        plsc.subcore_barrier()

        def body(i_vmem, o_vmem):
            pltpu.async_copy(table_sp.at[i_vmem.at[0]], o_vmem, sem).wait()

        pltpu.emit_pipeline(
            body,
            grid=(m // CHUNK,),
            in_specs=[pl.BlockSpec((1, CHUNK), index_map=lambda i: (0, i))],
            out_specs=[pl.BlockSpec((CHUNK, EMB), index_map=lambda i: (i, 0))],
            core_axis_name=("core", "subcore"),
            dimension_semantics=(pltpu.PARALLEL,),
        )(i_hbm, o_hbm)

    return k(table, idx2d)


def _sc_scatter_add(streams, zeros_init):
    nstream = len(streams)

    @functools.partial(
        pl.kernel,
        out_type=jax.ShapeDtypeStruct((2, NPAD, EMB), jnp.float32),
        mesh=_vector_mesh,
        scratch_types=[pltpu.VMEM_SHARED((NPAD, EMB), jnp.float32)],
    )
    def k(*args):
        refs = args[:2 * nstream]
        z_hbm = args[2 * nstream]
        out_hbm = args[2 * nstream + 1]
        acc = args[2 * nstream + 2]
        cid = lax.axis_index("core")
        sid = lax.axis_index("subcore")

        @pl.when(sid == 0)
        def _():
            pltpu.sync_copy(z_hbm, acc)

        plsc.subcore_barrier()

        def body(m_vmem, i_vmem):
            pltpu.sync_copy(m_vmem, acc.at[i_vmem.at[0]], add=True)

        for s in range(nstream):
            m_hbm, i_hbm = refs[s], refs[nstream + s]
            pltpu.emit_pipeline(
                body,
                grid=(m_hbm.shape[0] // CHUNK,),
                in_specs=[
                    pl.BlockSpec((CHUNK, EMB), index_map=lambda i: (i, 0)),
                    pl.BlockSpec((1, CHUNK), index_map=lambda i: (0, i)),
                ],
                out_specs=[],
                core_axis_name=("core", "subcore"),
                dimension_semantics=(pltpu.PARALLEL,),
            )(m_hbm, i_hbm)

        plsc.subcore_barrier()

        @pl.when(sid == 0)
        def _():
            pltpu.sync_copy(acc, out_hbm.at[cid])

    msgs = [s[0] for s in streams]
    idxs = [s[1] for s in streams]
    return k(*msgs, *idxs, zeros_init)


def _tc_binary_msgs(g, w1a, w1b, b1, w2, b2, blk=2048):
    nblk = H // blk

    def body(ge_ref, go_ref, w1a_ref, w1b_ref, b1_ref, w2_ref, b2_ref,
             me_ref, mo_ref):
        ge = ge_ref[...]
        go = go_ref[...]
        h = jnp.dot(ge.astype(jnp.bfloat16), w1a_ref[...],
                    preferred_element_type=jnp.float32)
        h += jnp.dot(go.astype(jnp.bfloat16), w1b_ref[...],
                     preferred_element_type=jnp.float32)
        h = jnp.maximum(h + b1_ref[...], 0.0).astype(jnp.bfloat16)
        y = jnp.dot(h, w2_ref[...], preferred_element_type=jnp.float32)
        y += b2_ref[...]
        me_ref[...] = ge + y[:, :EMB]
        mo_ref[...] = go + y[:, EMB:]

    out_shape = [jax.ShapeDtypeStruct((H, EMB), jnp.float32)] * 2
    return pl.pallas_call(
        body,
        grid=(nblk,),
        in_specs=[
            pl.BlockSpec((blk, EMB), lambda i: (i, 0)),
            pl.BlockSpec((blk, EMB), lambda i, nb=nblk: (i + nb, 0)),
            pl.BlockSpec((EMB, 2 * EMB), lambda i: (0, 0)),
            pl.BlockSpec((EMB, 2 * EMB), lambda i: (0, 0)),
            pl.BlockSpec((1, 2 * EMB), lambda i: (0, 0)),
            pl.BlockSpec((2 * EMB, 2 * EMB), lambda i: (0, 0)),
            pl.BlockSpec((1, 2 * EMB), lambda i: (0, 0)),
        ],
        out_specs=[
            pl.BlockSpec((blk, EMB), lambda i: (i, 0)),
            pl.BlockSpec((blk, EMB), lambda i: (i, 0)),
        ],
        out_shape=out_shape,
    )(g, g, w1a, w1b, b1, w2, b2)


def _tc_unary_msgs(g, w1, b1, w2, b2, blk=2048):
    nblk = NU_P // blk

    def body(gu_ref, w1_ref, b1_ref, w2_ref, b2_ref, mu_ref):
        gu = gu_ref[...]
        h = jnp.dot(gu.astype(jnp.bfloat16), w1_ref[...],
                    preferred_element_type=jnp.float32)
        h = jnp.maximum(h + b1_ref[...], 0.0).astype(jnp.bfloat16)
        y = jnp.dot(h, w2_ref[...], preferred_element_type=jnp.float32)
        mu_ref[...] = gu + y + b2_ref[...]

    return pl.pallas_call(
        body,
        grid=(nblk,),
        in_specs=[
            pl.BlockSpec((blk, EMB), lambda i: (i, 0)),
            pl.BlockSpec((EMB, EMB), lambda i: (0, 0)),
            pl.BlockSpec((1, EMB), lambda i: (0, 0)),
            pl.BlockSpec((EMB, EMB), lambda i: (0, 0)),
            pl.BlockSpec((1, EMB), lambda i: (0, 0)),
        ],
        out_specs=pl.BlockSpec((blk, EMB), lambda i: (i, 0)),
        out_shape=jax.ShapeDtypeStruct((NU_P, EMB), jnp.float32),
    )(g, w1, b1, w2, b2)


def _tc_update(p1, p2, p3, node_emb, w1, b1, w2, b2, blk=1000):
    nblk = N // blk

    def body(a0, a1, b0, b1r, c0, c1, emb_ref, w1_ref, b1_ref, w2_ref,
             b2_ref, o_ref):
        s = a0[0] + a1[0] + b0[0] + b1r[0] + c0[0] + c1[0]
        x = jnp.concatenate([s, emb_ref[...]], axis=1).astype(jnp.bfloat16)
        h = jnp.dot(x, w1_ref[...], preferred_element_type=jnp.float32)
        h = jnp.maximum(h + b1_ref[...], 0.0).astype(jnp.bfloat16)
        y = jnp.dot(h, w2_ref[...], preferred_element_type=jnp.float32)
        o_ref[...] = y + b2_ref[...]

    part_spec0 = pl.BlockSpec((1, blk, EMB), lambda i: (0, i, 0))
    part_spec1 = pl.BlockSpec((1, blk, EMB), lambda i: (1, i, 0))
    return pl.pallas_call(
        body,
        grid=(nblk,),
        in_specs=[
            part_spec0, part_spec1,
            part_spec0, part_spec1,
            part_spec0, part_spec1,
            pl.BlockSpec((blk, EMB), lambda i: (i, 0)),
            pl.BlockSpec((2 * EMB, 2 * EMB), lambda i: (0, 0)),
            pl.BlockSpec((1, 2 * EMB), lambda i: (0, 0)),
            pl.BlockSpec((2 * EMB, EMB), lambda i: (0, 0)),
            pl.BlockSpec((1, EMB), lambda i: (0, 0)),
        ],
        out_specs=pl.BlockSpec((blk, EMB), lambda i: (i, 0)),
        out_shape=jax.ShapeDtypeStruct((N, EMB), jnp.float32),
    )(p1, p1, p2, p2, p3, p3, node_emb, w1, b1, w2, b2)


def kernel(node_embeddings, rel_binary, rel_unary,
           Wb1, bb1, Wb2, bb2,
           Wu1, bu1, Wu2, bu2,
           Wup1, bup1, Wup2, bup2):
    bf = jnp.bfloat16
    idx_be = rel_binary[0::2]
    idx_bo = rel_binary[1::2]

    pad_b = 2 * H - NE
    pad_u = NU_P - NU
    zero_b = jnp.zeros((pad_b,), jnp.int32)
    zero_u = jnp.zeros((pad_u,), jnp.int32)
    dummy_b = jnp.full((pad_b,), N, jnp.int32)
    dummy_u = jnp.full((pad_u,), N, jnp.int32)

    ig1 = jnp.concatenate([idx_be[:H], idx_bo[:H]]).reshape(1, 2 * H)
    ie1 = idx_be[:H].reshape(1, H)
    io1 = idx_bo[:H].reshape(1, H)
    ig2 = jnp.concatenate(
        [idx_be[H:], zero_b, idx_bo[H:], zero_b]).reshape(1, 2 * H)
    ie2 = jnp.concatenate([idx_be[H:], dummy_b]).reshape(1, H)
    io2 = jnp.concatenate([idx_bo[H:], dummy_b]).reshape(1, H)
    igu = jnp.concatenate([rel_unary, zero_u]).reshape(1, NU_P)
    iu = jnp.concatenate([rel_unary, dummy_u]).reshape(1, NU_P)

    gu = _sc_gather(node_embeddings, igu)
    g1 = _sc_gather(node_embeddings, ig1)
    g2 = _sc_gather(node_embeddings, ig2)

    w1a, w1b = Wb1[:EMB].astype(bf), Wb1[EMB:].astype(bf)
    bb1r, bb2r = bb1.reshape(1, -1), bb2.reshape(1, -1)
    wb2 = Wb2.astype(bf)
    mu = _tc_unary_msgs(
        gu, Wu1.astype(bf), bu1.reshape(1, -1), Wu2.astype(bf),
        bu2.reshape(1, -1))
    me1, mo1 = _tc_binary_msgs(g1, w1a, w1b, bb1r, wb2, bb2r)
    me2, mo2 = _tc_binary_msgs(g2, w1a, w1b, bb1r, wb2, bb2r)

    zeros_init = jnp.zeros((NPAD, EMB), jnp.float32)
    p3 = _sc_scatter_add(((mu, iu),), zeros_init)
    p1 = _sc_scatter_add(((me1, ie1), (mo1, io1)), zeros_init)
    p2 = _sc_scatter_add(((me2, ie2), (mo2, io2)), zeros_init)

    return _tc_update(
        p1, p2, p3, node_embeddings, Wup1.astype(bf), bup1.reshape(1, -1),
        Wup2.astype(bf), bup2.reshape(1, -1))
